# baseline (speedup 1.0000x reference)
import jax
import jax.numpy as jnp
from jax.experimental import pallas as pl
from jax.experimental.pallas import tpu as pltpu

_MT = 512


def _chamfer_body(nj, scale, x_ref, yt_ref, out_ref, minl_ref):
    b = pl.program_id(0)
    j = pl.program_id(1)

    xb = x_ref[0]
    ytb = yt_ref[0]

    xr = xb.astype(jnp.bfloat16).astype(jnp.float32)
    yr = ytb.astype(jnp.bfloat16).astype(jnp.float32)

    xx = jnp.sum(xb * xb, axis=1, keepdims=True)
    yy = jnp.sum(ytb * ytb, axis=0, keepdims=True)

    x0 = xr[:, 0:1]
    x1 = xr[:, 1:2]
    x2 = xr[:, 2:3]
    y0 = yr[0:1, :]
    y1 = yr[1:2, :]
    y2 = yr[2:3, :]

    t = x0 * y0 + x1 * y1 + x2 * y2
    d = (xx + yy) - (t + t)

    @pl.when(j == 0)
    def _():
        minl_ref[...] = d

    @pl.when(j > 0)
    def _():
        minl_ref[...] = jnp.minimum(minl_ref[...], d)

    @pl.when((b == 0) & (j == 0))
    def _():
        out_ref[...] = jnp.zeros((1, 1), jnp.float32)

    minr = jnp.min(d, axis=0, keepdims=True)
    out_ref[...] += jnp.sum(minr, axis=1, keepdims=True) * scale

    @pl.when(j == nj - 1)
    def _():
        minl = jnp.min(minl_ref[...], axis=1, keepdims=True)
        out_ref[...] += jnp.sum(minl, axis=0, keepdims=True) * scale


def kernel(x, y):
    B, N, D = x.shape
    M = y.shape[1]
    nj = M // _MT
    scale = 1.0 / (B * N)

    yt = jnp.swapaxes(y, 1, 2)

    import functools
    body = functools.partial(_chamfer_body, nj, scale)

    out = pl.pallas_call(
        body,
        grid=(B, nj),
        in_specs=[
            pl.BlockSpec((1, N, D), lambda b, j: (b, 0, 0)),
            pl.BlockSpec((1, D, _MT), lambda b, j: (b, 0, j)),
        ],
        out_specs=pl.BlockSpec((1, 1), lambda b, j: (0, 0)),
        out_shape=jax.ShapeDtypeStruct((1, 1), jnp.float32),
        scratch_shapes=[pltpu.VMEM((N, _MT), jnp.float32)],
    )(x, yt)
    return out[0, 0]
